# baseline (speedup 1.0000x reference)
import functools

import jax
import jax.numpy as jnp
from jax import lax
from jax.experimental import pallas as pl
from jax.experimental.pallas import tpu as pltpu
from jax.experimental.pallas import tpu_sc as plsc

D_MODEL = 768
SEQ_LEN = 2048
BATCH = 4

NUM_WORKERS = 32
T_PER_W = SEQ_LEN // NUM_WORKERS
WIN = 16
NWIN = T_PER_W // WIN
VECS = D_MODEL // 16

_mesh = plsc.VectorSubcoreMesh(
    core_axis_name="c", subcore_axis_name="s", num_cores=2, num_subcores=16
)

_BUF = pltpu.VMEM((WIN, D_MODEL), jnp.float32)


@functools.partial(
    pl.kernel,
    out_type=jax.ShapeDtypeStruct((BATCH * SEQ_LEN, D_MODEL), jnp.float32),
    mesh=_mesh,
    scratch_types=[
        pltpu.VMEM((BATCH, T_PER_W), jnp.int32),
        [[_BUF for _ in range(BATCH)] for _ in range(2)],
        [_BUF, _BUF],
        [pltpu.SemaphoreType.DMA for _ in range(2)],
        [pltpu.SemaphoreType.DMA for _ in range(2)],
    ],
)
def _embed_sc(idx_hbm, w_hbm, pos_hbm, out_hbm, idx_v, gbufs, pbufs,
              ld_sems, st_sems):
    wid = lax.axis_index("s") * 2 + lax.axis_index("c")
    t0 = wid * T_PER_W

    idx_descs = [
        pltpu.async_copy(idx_hbm.at[b, pl.ds(t0, T_PER_W)], idx_v.at[b],
                         ld_sems[0])
        for b in range(BATCH)]
    for d in idx_descs:
        d.wait()

    def loads(w, p):
        descs = [pltpu.make_async_copy(
            w_hbm.at[idx_v.at[b, pl.ds(w * WIN, WIN)]], gbufs[p][b],
            ld_sems[p]) for b in range(BATCH)]
        descs.append(pltpu.make_async_copy(
            pos_hbm.at[pl.ds(t0 + w * WIN, WIN)], pbufs[p], ld_sems[p]))
        return descs

    def stores(w, p):
        return [pltpu.make_async_copy(
            gbufs[p][b],
            out_hbm.at[pl.ds(b * SEQ_LEN + t0 + w * WIN, WIN)],
            st_sems[p]) for b in range(BATCH)]

    def add_window(p):
        gb = gbufs[p]
        pb = pbufs[p]

        @plsc.parallel_loop(0, VECS, unroll=2)
        def _add_col(v):
            sl = pl.ds(v * 16, 16)
            for r in range(WIN):
                pvec = pb[r, sl]
                for b in range(BATCH):
                    plsc.addupdate(gb[b].at[r, sl], pvec)

    def win_step(w, p):
        @pl.when(w + 1 < NWIN)
        def _prefetch():
            @pl.when(w >= 1)
            def _drain():
                for d in stores(w - 1, 1 - p):
                    d.wait()
            for d in loads(w + 1, 1 - p):
                d.start()
        for d in loads(w, p):
            d.wait()
        add_window(p)
        for d in stores(w, p):
            d.start()

    for d in loads(0, 0):
        d.start()

    @pl.loop(0, NWIN // 2)
    def _pair(j):
        w0 = 2 * j
        win_step(w0, 0)
        win_step(w0 + 1, 1)

    for w, p in ((NWIN - 2, 0), (NWIN - 1, 1)):
        for d in stores(w, p):
            d.wait()


def kernel(x, W, pos_embd):
    idx = x if x.dtype == jnp.int32 else x.astype(jnp.int32)
    out = _embed_sc(idx, W, pos_embd)
    return out.reshape(BATCH, SEQ_LEN, D_MODEL)


# speedup vs baseline: 1.0987x; 1.0987x over previous
import functools

import jax
import jax.numpy as jnp
from jax import lax
from jax.experimental import pallas as pl
from jax.experimental.pallas import tpu as pltpu
from jax.experimental.pallas import tpu_sc as plsc

D_MODEL = 768
SEQ_LEN = 2048
BATCH = 4

NUM_WORKERS = 32
T_PER_W = SEQ_LEN // NUM_WORKERS
WIN = 8
NWIN = T_PER_W // WIN
VECS = D_MODEL // 16

_mesh = plsc.VectorSubcoreMesh(
    core_axis_name="c", subcore_axis_name="s", num_cores=2, num_subcores=16
)

_BUF = pltpu.VMEM((WIN, D_MODEL), jnp.float32)


@functools.partial(
    pl.kernel,
    out_type=jax.ShapeDtypeStruct((BATCH * SEQ_LEN, D_MODEL), jnp.float32),
    mesh=_mesh,
    scratch_types=[
        pltpu.VMEM((BATCH, T_PER_W), jnp.int32),
        [[_BUF for _ in range(BATCH)] for _ in range(2)],
        [_BUF, _BUF],
        [pltpu.SemaphoreType.DMA for _ in range(2)],
        [pltpu.SemaphoreType.DMA for _ in range(2)],
    ],
)
def _embed_sc(idx_hbm, w_hbm, pos_hbm, out_hbm, idx_v, gbufs, pbufs,
              ld_sems, st_sems):
    wid = lax.axis_index("s") * 2 + lax.axis_index("c")
    t0 = wid * T_PER_W

    idx_descs = [
        pltpu.async_copy(idx_hbm.at[b, pl.ds(t0, T_PER_W)], idx_v.at[b],
                         ld_sems[0])
        for b in range(BATCH)]
    for d in idx_descs:
        d.wait()

    def loads(w, p):
        descs = [pltpu.make_async_copy(
            w_hbm.at[idx_v.at[b, pl.ds(w * WIN, WIN)]], gbufs[p][b],
            ld_sems[p]) for b in range(BATCH)]
        descs.append(pltpu.make_async_copy(
            pos_hbm.at[pl.ds(t0 + w * WIN, WIN)], pbufs[p], ld_sems[p]))
        return descs

    def stores(w, p):
        return [pltpu.make_async_copy(
            gbufs[p][b],
            out_hbm.at[pl.ds(b * SEQ_LEN + t0 + w * WIN, WIN)],
            st_sems[p]) for b in range(BATCH)]

    def add_window(p):
        gb = gbufs[p]
        pb = pbufs[p]

        @plsc.parallel_loop(0, VECS, unroll=1)
        def _add_col(v):
            sl = pl.ds(v * 16, 16)
            for r in range(WIN):
                pvec = pb[r, sl]
                for b in range(BATCH):
                    plsc.addupdate(gb[b].at[r, sl], pvec)

    def win_step(w, p):
        @pl.when(w + 1 < NWIN)
        def _prefetch():
            @pl.when(w >= 1)
            def _drain():
                for d in stores(w - 1, 1 - p):
                    d.wait()
            for d in loads(w + 1, 1 - p):
                d.start()
        for d in loads(w, p):
            d.wait()
        add_window(p)
        for d in stores(w, p):
            d.start()

    for d in loads(0, 0):
        d.start()

    @pl.loop(0, NWIN // 2)
    def _pair(j):
        w0 = 2 * j
        win_step(w0, 0)
        win_step(w0 + 1, 1)

    for w, p in ((NWIN - 2, 0), (NWIN - 1, 1)):
        for d in stores(w, p):
            d.wait()


def kernel(x, W, pos_embd):
    idx = x if x.dtype == jnp.int32 else x.astype(jnp.int32)
    out = _embed_sc(idx, W, pos_embd)
    return out.reshape(BATCH, SEQ_LEN, D_MODEL)
